# baseline (speedup 1.0000x reference)
import functools

import jax
import jax.numpy as jnp
from jax import lax
from jax.experimental import pallas as pl
from jax.experimental.pallas import tpu as pltpu
from jax.experimental.pallas import tpu_sc as plsc

_NC = 2
_NS = 16
_NW = _NC * _NS


def _mlp_body(x_ref, w1_ref, b1_ref, w2_ref, b2_ref, w3_ref, b3_ref, o_ref):
    bf = jnp.bfloat16
    h = jnp.dot(x_ref[...].astype(bf), w1_ref[...].astype(bf),
                preferred_element_type=jnp.float32)
    h = jnp.maximum(h + b1_ref[...], 0.0)
    h = jnp.dot(h.astype(bf), w2_ref[...].astype(bf),
                preferred_element_type=jnp.float32) + b2_ref[...]
    o_ref[...] = jnp.dot(h.astype(bf), w3_ref[...].astype(bf),
                         preferred_element_type=jnp.float32) + b3_ref[...]


def _mlp(x, W1, b1, W2, b2, W3, b3):
    n, d = x.shape
    h = W1.shape[1]
    h2 = W2.shape[1]
    out = W3.shape[1]
    blk = n
    grid = pl.cdiv(n, blk)
    return pl.pallas_call(
        _mlp_body,
        grid=(grid,),
        in_specs=[
            pl.BlockSpec((blk, d), lambda i: (i, 0)),
            pl.BlockSpec((d, h), lambda i: (0, 0)),
            pl.BlockSpec((1, h), lambda i: (0, 0)),
            pl.BlockSpec((h, h2), lambda i: (0, 0)),
            pl.BlockSpec((1, h2), lambda i: (0, 0)),
            pl.BlockSpec((h2, out), lambda i: (0, 0)),
            pl.BlockSpec((1, out), lambda i: (0, 0)),
        ],
        out_specs=pl.BlockSpec((blk, out), lambda i: (i, 0)),
        out_shape=jax.ShapeDtypeStruct((n, out), jnp.float32),
    )(x, W1, b1[None, :], W2, b2[None, :], W3, b3[None, :])


def _round_up(v, m):
    return (v + m - 1) // m * m


def _gather_pairs(xdnn, x, idx_l, idx_r):
    p = idx_l.shape[0]
    n, d = x.shape
    cpw = _round_up(pl.cdiv(p, _NW), 8)
    chunk = 64
    nb = 5
    nchunks = pl.cdiv(cpw, chunk)
    spw = _round_up(pl.cdiv(n, _NS), 8)
    mesh = plsc.VectorSubcoreMesh(
        core_axis_name="c", subcore_axis_name="s",
        num_cores=_NC, num_subcores=_NS)

    @functools.partial(
        pl.kernel,
        out_type=[
            jax.ShapeDtypeStruct((2 * p, d), jnp.float32),
            jax.ShapeDtypeStruct((2 * p, d), jnp.float32),
        ],
        mesh=mesh,
        scratch_types=[
            pltpu.VMEM((cpw,), jnp.int32),
            pltpu.VMEM((cpw,), jnp.int32),
            pltpu.VMEM_SHARED((n, d), jnp.float32),
        ] + [pltpu.VMEM((chunk, d), jnp.float32) for _ in range(nb)]
          + [pltpu.SemaphoreType.DMA for _ in range(2 * nb)],
    )
    def k(xdnn_hbm, x_hbm, idxl_hbm, idxr_hbm, emb_out, feat_out,
          idxl_v, idxr_v, staged, *bufsems):
        bufs = bufsems[:nb]
        gsems = bufsems[nb:2 * nb]
        ssems = bufsems[2 * nb:]
        cid = lax.axis_index("c")
        sid = lax.axis_index("s")
        wid = sid * _NC + cid
        base = jnp.minimum(wid * cpw, p - cpw)
        pltpu.sync_copy(idxl_hbm.at[pl.ds(base, cpw)], idxl_v)
        pltpu.sync_copy(idxr_hbm.at[pl.ds(base, cpw)], idxr_v)
        sbase = jnp.minimum(sid * spw, n - spw)

        def off(j):
            return jnp.minimum(j * chunk, cpw - chunk)

        def stage(table_hbm):
            pltpu.sync_copy(table_hbm.at[pl.ds(sbase, spw)],
                            staged.at[pl.ds(sbase, spw)])

        def run_phase(out, prev_puts):
            for dsc in prev_puts:
                dsc.wait()
            units = []
            for idx_v, row0 in ((idxl_v, 0), (idxr_v, p)):
                for j in range(nchunks):
                    units.append((idx_v, row0, j))
            nu = len(units)

            def gather(i):
                idx_v, _, j = units[i]
                return pltpu.async_copy(
                    staged.at[idx_v.at[pl.ds(off(j), chunk)]],
                    bufs[i % nb], gsems[i % nb])

            def scatter(i):
                _, row0, j = units[i]
                return pltpu.async_copy(
                    bufs[i % nb],
                    out.at[pl.ds(row0 + base + off(j), chunk)],
                    ssems[i % nb])

            gets = [gather(i) for i in range(min(nb - 1, nu))]
            puts = []
            for i in range(nu):
                if i + nb - 1 < nu:
                    if i - 1 >= 0:
                        puts[i - 1].wait()
                    gets.append(gather(i + nb - 1))
                gets[i].wait()
                puts.append(scatter(i))
            return puts[max(0, nu - nb):]

        stage(xdnn_hbm)
        plsc.subcore_barrier()
        tail1 = run_phase(emb_out, [])
        plsc.subcore_barrier()
        stage(x_hbm)
        plsc.subcore_barrier()
        tail2 = run_phase(feat_out, tail1)
        for dsc in tail2:
            dsc.wait()

    emb, feat = k(xdnn, x, idx_l, idx_r)
    return emb.reshape(2, p, d), feat.reshape(2, p, d)


def kernel(x, edge_index, pair_idxs_left, pair_idxs_right, y,
           W_lin, b_lin, W_gat, a_src, a_dst, b_gat,
           W1, b1, W2, b2, W3, b3):
    x_dnn = _mlp(x, W1, b1, W2, b2, W3, b3)
    pair_embeddings, pair_features = _gather_pairs(
        x_dnn, x, pair_idxs_left, pair_idxs_right)
    return (pair_embeddings, pair_features, y)


# speedup vs baseline: 1.0135x; 1.0135x over previous
import functools

import jax
import jax.numpy as jnp
from jax import lax
from jax.experimental import pallas as pl
from jax.experimental.pallas import tpu as pltpu
from jax.experimental.pallas import tpu_sc as plsc

_NC = 2
_NS = 16
_NW = _NC * _NS


def _mlp_body(x_ref, w1_ref, b1_ref, w2_ref, b2_ref, w3_ref, b3_ref, o_ref):
    h = jnp.dot(x_ref[...], w1_ref[...], preferred_element_type=jnp.float32)
    h = jnp.maximum(h + b1_ref[...], 0.0)
    h = jnp.dot(h, w2_ref[...], preferred_element_type=jnp.float32) + b2_ref[...]
    o_ref[...] = jnp.dot(h, w3_ref[...], preferred_element_type=jnp.float32) + b3_ref[...]


def _mlp(x, W1, b1, W2, b2, W3, b3):
    n, d = x.shape
    h = W1.shape[1]
    h2 = W2.shape[1]
    out = W3.shape[1]
    blk = n
    grid = pl.cdiv(n, blk)
    return pl.pallas_call(
        _mlp_body,
        grid=(grid,),
        in_specs=[
            pl.BlockSpec((blk, d), lambda i: (i, 0)),
            pl.BlockSpec((d, h), lambda i: (0, 0)),
            pl.BlockSpec((1, h), lambda i: (0, 0)),
            pl.BlockSpec((h, h2), lambda i: (0, 0)),
            pl.BlockSpec((1, h2), lambda i: (0, 0)),
            pl.BlockSpec((h2, out), lambda i: (0, 0)),
            pl.BlockSpec((1, out), lambda i: (0, 0)),
        ],
        out_specs=pl.BlockSpec((blk, out), lambda i: (i, 0)),
        out_shape=jax.ShapeDtypeStruct((n, out), jnp.float32),
    )(x, W1, b1[None, :], W2, b2[None, :], W3, b3[None, :])


def _round_up(v, m):
    return (v + m - 1) // m * m


def _gather_pairs(xdnn, x, idx_l, idx_r):
    p = idx_l.shape[0]
    n, d = x.shape
    cpw = _round_up(pl.cdiv(p, _NW), 8)
    chunk = 64
    nb = 5
    nchunks = pl.cdiv(cpw, chunk)
    spw = _round_up(pl.cdiv(n, _NS), 8)
    mesh = plsc.VectorSubcoreMesh(
        core_axis_name="c", subcore_axis_name="s",
        num_cores=_NC, num_subcores=_NS)

    @functools.partial(
        pl.kernel,
        out_type=[
            jax.ShapeDtypeStruct((2 * p, d), jnp.float32),
            jax.ShapeDtypeStruct((2 * p, d), jnp.float32),
        ],
        mesh=mesh,
        scratch_types=[
            pltpu.VMEM((cpw,), jnp.int32),
            pltpu.VMEM((cpw,), jnp.int32),
            pltpu.VMEM_SHARED((n, d), jnp.float32),
        ] + [pltpu.VMEM((chunk, d), jnp.float32) for _ in range(nb)]
          + [pltpu.SemaphoreType.DMA for _ in range(2 * nb)],
    )
    def k(xdnn_hbm, x_hbm, idxl_hbm, idxr_hbm, emb_out, feat_out,
          idxl_v, idxr_v, staged, *bufsems):
        bufs = bufsems[:nb]
        gsems = bufsems[nb:2 * nb]
        ssems = bufsems[2 * nb:]
        cid = lax.axis_index("c")
        sid = lax.axis_index("s")
        wid = sid * _NC + cid
        base = jnp.minimum(wid * cpw, p - cpw)
        idx_loads = [
            pltpu.async_copy(idxl_hbm.at[pl.ds(base, cpw)], idxl_v, gsems[0]),
            pltpu.async_copy(idxr_hbm.at[pl.ds(base, cpw)], idxr_v, gsems[1]),
        ]
        sbase = jnp.minimum(sid * spw, n - spw)

        def off(j):
            return jnp.minimum(j * chunk, cpw - chunk)

        def stage(table_hbm):
            pltpu.sync_copy(table_hbm.at[pl.ds(sbase, spw)],
                            staged.at[pl.ds(sbase, spw)])

        def run_phase(out, prev_puts):
            for dsc in prev_puts:
                dsc.wait()
            units = []
            for idx_v, row0 in ((idxl_v, 0), (idxr_v, p)):
                for j in range(nchunks):
                    units.append((idx_v, row0, j))
            nu = len(units)

            def gather(i):
                idx_v, _, j = units[i]
                return pltpu.async_copy(
                    staged.at[idx_v.at[pl.ds(off(j), chunk)]],
                    bufs[i % nb], gsems[i % nb])

            def scatter(i):
                _, row0, j = units[i]
                return pltpu.async_copy(
                    bufs[i % nb],
                    out.at[pl.ds(row0 + base + off(j), chunk)],
                    ssems[i % nb])

            gets = [gather(i) for i in range(min(nb - 1, nu))]
            puts = []
            for i in range(nu):
                if i + nb - 1 < nu:
                    if i - 1 >= 0:
                        puts[i - 1].wait()
                    gets.append(gather(i + nb - 1))
                gets[i].wait()
                puts.append(scatter(i))
            return puts[max(0, nu - nb):]

        stage(xdnn_hbm)
        for dsc in idx_loads:
            dsc.wait()
        plsc.subcore_barrier()
        tail1 = run_phase(emb_out, [])
        plsc.subcore_barrier()
        stage(x_hbm)
        plsc.subcore_barrier()
        tail2 = run_phase(feat_out, tail1)
        for dsc in tail2:
            dsc.wait()

    emb, feat = k(xdnn, x, idx_l, idx_r)
    return emb.reshape(2, p, d), feat.reshape(2, p, d)


def kernel(x, edge_index, pair_idxs_left, pair_idxs_right, y,
           W_lin, b_lin, W_gat, a_src, a_dst, b_gat,
           W1, b1, W2, b2, W3, b3):
    x_dnn = _mlp(x, W1, b1, W2, b2, W3, b3)
    pair_embeddings, pair_features = _gather_pairs(
        x_dnn, x, pair_idxs_left, pair_idxs_right)
    return (pair_embeddings, pair_features, y)
